# baseline (speedup 1.0000x reference)
import functools

import jax
import jax.numpy as jnp
from jax import lax
from jax.experimental import pallas as pl
from jax.experimental.pallas import tpu as pltpu
from jax.experimental.pallas import tpu_sc as plsc

B = 16384
K = 32
L = 16

_info = plsc.get_sparse_core_info()
NC = _info.num_cores
NS = _info.num_subcores
NW = NC * NS
BPW = B // NW
NG = BPW // L
CH = 128

_mesh = plsc.VectorSubcoreMesh(core_axis_name="c", subcore_axis_name="s")


@functools.partial(
    pl.kernel,
    mesh=_mesh,
    compiler_params=pltpu.CompilerParams(
        needs_layout_passes=False,
        use_tc_tiling_on_sc=False,
    ),
    out_type=jax.ShapeDtypeStruct((B,), jnp.float32),
    scratch_types=[
        pltpu.VMEM((BPW,), jnp.int32),
        pltpu.VMEM((BPW,), jnp.int32),
        pltpu.VMEM((BPW,), jnp.int32),
        pltpu.VMEM((BPW,), jnp.int32),
        pltpu.VMEM((BPW, K), jnp.float32),
        pltpu.VMEM((BPW, K), jnp.float32),
        pltpu.VMEM((BPW, L), jnp.float32),
        pltpu.VMEM((BPW, L), jnp.float32),
        pltpu.VMEM((L,), jnp.float32),
        pltpu.VMEM((BPW,), jnp.float32),
        pltpu.SemaphoreType.DMA,
    ],
)
def _sc_predict(u_hbm, i_hbm, user_hbm, bu_hbm, movie_hbm, bm_hbm, mean_hbm,
                out_hbm, idx_u, idx_i, hi_u, hi_i, ue_v, me_v, bu_v, bm_v,
                mean_v, out_v, sem):
    wid = lax.axis_index("s") * NC + lax.axis_index("c")
    base = wid * BPW

    pltpu.sync_copy(u_hbm.at[pl.ds(base, BPW)], idx_u)
    pltpu.sync_copy(i_hbm.at[pl.ds(base, BPW)], idx_i)
    pltpu.sync_copy(mean_hbm, mean_v)

    copies = []
    for c in range(BPW // CH):
        s = pl.ds(c * CH, CH)
        copies.append(pltpu.async_copy(user_hbm.at[idx_u.at[s]], ue_v.at[s], sem))
        copies.append(pltpu.async_copy(movie_hbm.at[idx_i.at[s]], me_v.at[s], sem))

    def hi_body(c, _):
        s = pl.ds(pl.multiple_of(c * L, L), L)
        hi_u[s] = lax.shift_right_logical(idx_u[s], 4)
        hi_i[s] = lax.shift_right_logical(idx_i[s], 4)
        return 0

    lax.fori_loop(0, NG, hi_body, 0)

    for c in range(BPW // CH):
        s = pl.ds(c * CH, CH)
        copies.append(pltpu.async_copy(bu_hbm.at[hi_u.at[s]], bu_v.at[s], sem))
        copies.append(pltpu.async_copy(bm_hbm.at[hi_i.at[s]], bm_v.at[s], sem))
    for c in copies:
        c.wait()

    mean = mean_v[...]
    lo_mask = jnp.full((L,), 15, jnp.int32)

    def group_body(g, _):
        s = pl.ds(pl.multiple_of(g * L, L), L)
        rid = g * L + lax.iota(jnp.int32, L)
        acc = jnp.zeros((L,), jnp.float32)
        for k in range(K):
            kk = jnp.full((L,), k, jnp.int32)
            uc = plsc.load_gather(ue_v, [rid, kk])
            mc = plsc.load_gather(me_v, [rid, kk])
            acc = acc + uc * mc
        bu = plsc.load_gather(bu_v, [rid, idx_u[s] & lo_mask])
        bm = plsc.load_gather(bm_v, [rid, idx_i[s] & lo_mask])
        out_v[s] = acc + bu + bm + mean
        return 0

    lax.fori_loop(0, NG, group_body, 0)

    pltpu.sync_copy(out_v, out_hbm.at[pl.ds(base, BPW)])


def kernel(u, i, user, bias_user, movie, bias_movie, mean):
    user_rm = lax.optimization_barrier(user.reshape(-1)).reshape(user.shape)
    movie_rm = lax.optimization_barrier(movie.reshape(-1)).reshape(movie.shape)
    bu16 = bias_user.reshape(-1, L)
    bm16 = bias_movie.reshape(-1, L)
    mean_v = jnp.full((L,), mean, dtype=jnp.float32)
    return _sc_predict(u, i, user_rm, bu16, movie_rm, bm16, mean_v)
